# Initial kernel scaffold; baseline (speedup 1.0000x reference)
import jax
import jax.numpy as jnp
from jax.experimental import pallas as pl


def kernel(x, edge_index, Wl1, Wr1, b1, Wl2, Wr2, b2):
    raise NotImplementedError("write your pallas kernel here")


# speedup vs baseline: 3.3362x; 3.3362x over previous
import functools

import jax
import jax.numpy as jnp
from jax import lax
from jax.experimental import pallas as pl
from jax.experimental.pallas import tpu as pltpu
from jax.experimental.pallas import tpu_sc as plsc

N_NODES = 10000
D = 128
NC, NS = 2, 16
NW = NC * NS
NPAD = 10240
RPT = NPAD // NS
CHUNK = 128
EPW = 10240
NCHUNK = EPW // CHUNK
NE_PAD = NW * EPW


def _sc_aggregate_body(y_hbm, src_hbm, dst_hbm, psum_hbm, pdeg_hbm,
                       sidx_v, didx_v, rows_v, zdeg_v, ones_v,
                       acc_sh, deg_sh, sem):
    c = lax.axis_index("c")
    s = lax.axis_index("s")
    wid = c * NS + s

    zero16 = jnp.zeros((16,), jnp.float32)
    one16 = jnp.ones((16,), jnp.float32)

    def fill_zrow(i, carry):
        for k in range(D // 16):
            rows_v[i, pl.ds(k * 16, 16)] = zero16
        return carry
    lax.fori_loop(0, CHUNK, fill_zrow, 0)

    def fill_zdeg(i, carry):
        zdeg_v[pl.ds(i * 16, 16)] = zero16
        return carry
    lax.fori_loop(0, RPT // 16, fill_zdeg, 0)

    for k in range(CHUNK // 16):
        ones_v[pl.ds(k * 16, 16)] = one16

    base_r = s * RPT
    for j in range(RPT // CHUNK):
        pltpu.sync_copy(rows_v, acc_sh.at[pl.ds(base_r + j * CHUNK, CHUNK)])
    pltpu.sync_copy(zdeg_v, deg_sh.at[pl.ds(base_r, RPT)])
    plsc.subcore_barrier()

    pltpu.sync_copy(src_hbm.at[wid], sidx_v)
    pltpu.sync_copy(dst_hbm.at[wid], didx_v)

    def chunk_body(ci, carry):
        pltpu.async_copy(y_hbm.at[sidx_v.at[ci]], rows_v, sem).wait()
        pltpu.sync_copy(rows_v, acc_sh.at[didx_v.at[ci]], add=True)
        pltpu.sync_copy(ones_v, deg_sh.at[didx_v.at[ci]], add=True)
        return carry
    lax.fori_loop(0, NCHUNK, chunk_body, 0)

    plsc.subcore_barrier()
    pltpu.sync_copy(acc_sh.at[pl.ds(base_r, RPT)],
                    psum_hbm.at[c, pl.ds(base_r, RPT)])
    pltpu.sync_copy(deg_sh.at[pl.ds(base_r, RPT)],
                    pdeg_hbm.at[c, pl.ds(base_r, RPT)])


_sc_aggregate = functools.partial(
    pl.kernel,
    out_type=(jax.ShapeDtypeStruct((NC, NPAD, D), jnp.float32),
              jax.ShapeDtypeStruct((NC, NPAD), jnp.float32)),
    mesh=plsc.VectorSubcoreMesh(core_axis_name="c", subcore_axis_name="s"),
    scratch_types=[
        pltpu.VMEM((NCHUNK, CHUNK), jnp.int32),
        pltpu.VMEM((NCHUNK, CHUNK), jnp.int32),
        pltpu.VMEM((CHUNK, D), jnp.float32),
        pltpu.VMEM((RPT,), jnp.float32),
        pltpu.VMEM((CHUNK,), jnp.float32),
        pltpu.VMEM_SHARED((NPAD, D), jnp.float32),
        pltpu.VMEM_SHARED((NPAD,), jnp.float32),
        pltpu.SemaphoreType.DMA,
    ],
)(_sc_aggregate_body)


BR = 400
_DN = (((1,), (1,)), ((), ()))


def _tc_pre_body(x_ref, wl_ref, wr_ref, b_ref, y_ref, r_ref):
    xb = x_ref[...]
    y_ref[...] = lax.dot_general(xb, wl_ref[...], _DN,
                                 preferred_element_type=jnp.float32)
    r_ref[...] = lax.dot_general(xb, wr_ref[...], _DN,
                                 preferred_element_type=jnp.float32) + b_ref[...]


_tc_pre = pl.pallas_call(
    _tc_pre_body,
    grid=(N_NODES // BR,),
    in_specs=[
        pl.BlockSpec((BR, D), lambda i: (i, 0)),
        pl.BlockSpec((D, D), lambda i: (0, 0)),
        pl.BlockSpec((D, D), lambda i: (0, 0)),
        pl.BlockSpec((1, D), lambda i: (0, 0)),
    ],
    out_specs=[
        pl.BlockSpec((BR, D), lambda i: (i, 0)),
        pl.BlockSpec((BR, D), lambda i: (i, 0)),
    ],
    out_shape=[
        jax.ShapeDtypeStruct((N_NODES, D), jnp.float32),
        jax.ShapeDtypeStruct((N_NODES, D), jnp.float32),
    ],
)


def _tc_mid_body(p0_ref, p1_ref, d0_ref, d1_ref, r1_ref, wl_ref, wr_ref,
                 b_ref, y_ref, r_ref):
    deg = jnp.maximum(d0_ref[...] + d1_ref[...], 1.0)
    h = jnp.maximum((p0_ref[...] + p1_ref[...]) / deg + r1_ref[...], 0.0)
    y_ref[...] = lax.dot_general(h, wl_ref[...], _DN,
                                 preferred_element_type=jnp.float32)
    r_ref[...] = lax.dot_general(h, wr_ref[...], _DN,
                                 preferred_element_type=jnp.float32) + b_ref[...]


_tc_mid = pl.pallas_call(
    _tc_mid_body,
    grid=(N_NODES // BR,),
    in_specs=[
        pl.BlockSpec((BR, D), lambda i: (i, 0)),
        pl.BlockSpec((BR, D), lambda i: (i, 0)),
        pl.BlockSpec((BR, 1), lambda i: (i, 0)),
        pl.BlockSpec((BR, 1), lambda i: (i, 0)),
        pl.BlockSpec((BR, D), lambda i: (i, 0)),
        pl.BlockSpec((D, D), lambda i: (0, 0)),
        pl.BlockSpec((D, D), lambda i: (0, 0)),
        pl.BlockSpec((1, D), lambda i: (0, 0)),
    ],
    out_specs=[
        pl.BlockSpec((BR, D), lambda i: (i, 0)),
        pl.BlockSpec((BR, D), lambda i: (i, 0)),
    ],
    out_shape=[
        jax.ShapeDtypeStruct((N_NODES, D), jnp.float32),
        jax.ShapeDtypeStruct((N_NODES, D), jnp.float32),
    ],
)


def _tc_fin_body(p0_ref, p1_ref, d0_ref, d1_ref, r2_ref, o_ref):
    deg = jnp.maximum(d0_ref[...] + d1_ref[...], 1.0)
    o_ref[...] = (p0_ref[...] + p1_ref[...]) / deg + r2_ref[...]


_tc_fin = pl.pallas_call(
    _tc_fin_body,
    grid=(N_NODES // BR,),
    in_specs=[
        pl.BlockSpec((BR, D), lambda i: (i, 0)),
        pl.BlockSpec((BR, D), lambda i: (i, 0)),
        pl.BlockSpec((BR, 1), lambda i: (i, 0)),
        pl.BlockSpec((BR, 1), lambda i: (i, 0)),
        pl.BlockSpec((BR, D), lambda i: (i, 0)),
    ],
    out_specs=pl.BlockSpec((BR, D), lambda i: (i, 0)),
    out_shape=jax.ShapeDtypeStruct((N_NODES, D), jnp.float32),
)


def kernel(x, edge_index, Wl1, Wr1, b1, Wl2, Wr2, b2):
    src = edge_index[0].astype(jnp.int32)
    dst = edge_index[1].astype(jnp.int32)
    n_extra = NE_PAD - src.shape[0]
    pad_src = jnp.zeros((n_extra,), jnp.int32)
    pad_dst = N_NODES + (jnp.arange(n_extra, dtype=jnp.int32)
                         % (NPAD - N_NODES))
    src3 = jnp.concatenate([src, pad_src]).reshape(NW, NCHUNK, CHUNK)
    dst3 = jnp.concatenate([dst, pad_dst]).reshape(NW, NCHUNK, CHUNK)

    b1r = b1.reshape(1, D)
    b2r = b2.reshape(1, D)

    y1, r1 = _tc_pre(x, Wl1, Wr1, b1r)
    p1s, p1d = _sc_aggregate(y1, src3, dst3)
    y2, r2 = _tc_mid(p1s[0, :N_NODES], p1s[1, :N_NODES],
                     p1d[0, :N_NODES, None], p1d[1, :N_NODES, None],
                     r1, Wl2, Wr2, b2r)
    p2s, p2d = _sc_aggregate(y2, src3, dst3)
    out = _tc_fin(p2s[0, :N_NODES], p2s[1, :N_NODES],
                  p2d[0, :N_NODES, None], p2d[1, :N_NODES, None], r2)
    return out
